# baseline (speedup 1.0000x reference)
import functools

import jax
import jax.numpy as jnp
from jax import lax
from jax.experimental import pallas as pl
from jax.experimental.pallas import tpu as pltpu
from jax.experimental.pallas import tpu_sc as plsc

BATCH = 4096
VOCAB = 1000
DIM = 128
LANES = 16

_info = plsc.get_sparse_core_info()
_NC, _NS = _info.num_cores, _info.num_subcores
_NW = _NC * _NS
_B_PER_W = BATCH // _NW
_CHUNKS = (8, 24, 40, 56)
_NB = len(_CHUNKS)

_mesh = plsc.VectorSubcoreMesh(core_axis_name="c", subcore_axis_name="s")


@functools.partial(
    pl.kernel,
    mesh=_mesh,
    out_type=jax.ShapeDtypeStruct((BATCH, 2, DIM), jnp.float32),
    scratch_types=(
        [
            pltpu.VMEM((_B_PER_W,), jnp.int32),
            pltpu.VMEM((_B_PER_W,), jnp.int32),
            pltpu.VMEM((_B_PER_W, 1, DIM), jnp.float32),
            pltpu.VMEM((_B_PER_W, 1, DIM), jnp.float32),
            pltpu.VMEM((DIM,), jnp.float32),
            pltpu.VMEM((DIM,), jnp.float32),
        ]
        + [pltpu.SemaphoreType.DMA] * (4 * _NB + 1)
    ),
)
def _ctx_emb_kernel(sess_idx_hbm, subj_idx_hbm, sess_tab_hbm, subj_tab_hbm,
                    sess_flag_hbm, subj_flag_hbm, out_hbm,
                    idx_s, idx_b, rows_s, rows_b, flag_s, flag_b, *sems):
    sem_s = sems[0:_NB]
    sem_b = sems[_NB:2 * _NB]
    sem_os = sems[2 * _NB:3 * _NB]
    sem_ob = sems[3 * _NB:4 * _NB]
    sem_f = sems[4 * _NB]

    wid = lax.axis_index("s") * _NC + lax.axis_index("c")
    base = wid * _B_PER_W

    cp_fs = pltpu.async_copy(sess_flag_hbm, flag_s, sem_f)
    cp_fb = pltpu.async_copy(subj_flag_hbm, flag_b, sem_f)

    pltpu.sync_copy(sess_idx_hbm.at[pl.ds(base, _B_PER_W)], idx_s)
    pltpu.sync_copy(subj_idx_hbm.at[pl.ds(base, _B_PER_W)], idx_b)

    offs = [sum(_CHUNKS[:k]) for k in range(_NB)]
    cp_s = []
    cp_b = []
    for k, (off, n) in enumerate(zip(offs, _CHUNKS)):
        r = pl.ds(off, n)
        cp_s.append(pltpu.async_copy(
            sess_tab_hbm.at[idx_s.at[r]], rows_s.at[r], sem_s[k]))
        cp_b.append(pltpu.async_copy(
            subj_tab_hbm.at[idx_b.at[r]], rows_b.at[r], sem_b[k]))

    cp_fs.wait()
    cp_fb.wait()
    fs = [flag_s[pl.ds(i * LANES, LANES)] for i in range(DIM // LANES)]
    fb = [flag_b[pl.ds(i * LANES, LANES)] for i in range(DIM // LANES)]

    cp_o = []
    for k, (off, n) in enumerate(zip(offs, _CHUNKS)):
        r = pl.ds(off, n)
        ro = pl.ds(base + off, n)

        cp_s[k].wait()
        cp_b[k].wait()

        @plsc.parallel_loop(off, off + n, unroll=1)
        def _body(i):
            for c in range(DIM // LANES):
                d = pl.ds(c * LANES, LANES)
                plsc.addupdate(rows_s.at[i, 0, d], fs[c])
                plsc.addupdate(rows_b.at[i, 0, d], fb[c])

        cp_o.append(pltpu.async_copy(
            rows_s.at[r], out_hbm.at[ro, pl.ds(0, 1)], sem_os[k]))
        cp_o.append(pltpu.async_copy(
            rows_b.at[r], out_hbm.at[ro, pl.ds(1, 1)], sem_ob[k]))

    for cp in cp_o:
        cp.wait()


def kernel(session_idx, subject_idx, session_table, subject_table,
           session_flag, subject_flag):
    return _ctx_emb_kernel(
        session_idx, subject_idx,
        session_table.reshape(VOCAB, 1, DIM),
        subject_table.reshape(VOCAB, 1, DIM),
        session_flag, subject_flag)


# speedup vs baseline: 1.0230x; 1.0230x over previous
import functools

import jax
import jax.numpy as jnp
from jax import lax
from jax.experimental import pallas as pl
from jax.experimental.pallas import tpu as pltpu
from jax.experimental.pallas import tpu_sc as plsc

BATCH = 4096
VOCAB = 1000
DIM = 128
LANES = 16

_info = plsc.get_sparse_core_info()
_NC, _NS = _info.num_cores, _info.num_subcores
_NW = _NC * _NS
_B_PER_W = BATCH // _NW
_CHUNKS = (8, 24, 40, 56)
_NB = len(_CHUNKS)

_mesh = plsc.VectorSubcoreMesh(core_axis_name="c", subcore_axis_name="s")


@functools.partial(
    pl.kernel,
    mesh=_mesh,
    out_type=jax.ShapeDtypeStruct((BATCH, 2, DIM), jnp.float32),
    scratch_types=(
        [
            pltpu.VMEM((_B_PER_W,), jnp.int32),
            pltpu.VMEM((_B_PER_W,), jnp.int32),
            pltpu.VMEM((_B_PER_W, 1, DIM), jnp.float32),
            pltpu.VMEM((_B_PER_W, 1, DIM), jnp.float32),
            pltpu.VMEM((DIM,), jnp.float32),
            pltpu.VMEM((DIM,), jnp.float32),
        ]
        + [pltpu.SemaphoreType.DMA] * (4 * _NB + 3)
    ),
)
def _ctx_emb_kernel(sess_idx_hbm, subj_idx_hbm, sess_tab_hbm, subj_tab_hbm,
                    sess_flag_hbm, subj_flag_hbm, out_hbm,
                    idx_s, idx_b, rows_s, rows_b, flag_s, flag_b, *sems):
    sem_s = sems[0:_NB]
    sem_b = sems[_NB:2 * _NB]
    sem_os = sems[2 * _NB:3 * _NB]
    sem_ob = sems[3 * _NB:4 * _NB]
    sem_f = sems[4 * _NB]
    sem_is = sems[4 * _NB + 1]
    sem_ib = sems[4 * _NB + 2]

    wid = lax.axis_index("s") * _NC + lax.axis_index("c")
    base = wid * _B_PER_W

    cp_fs = pltpu.async_copy(sess_flag_hbm, flag_s, sem_f)
    cp_fb = pltpu.async_copy(subj_flag_hbm, flag_b, sem_f)
    cp_is = pltpu.async_copy(
        sess_idx_hbm.at[pl.ds(base, _B_PER_W)], idx_s, sem_is)
    cp_ib = pltpu.async_copy(
        subj_idx_hbm.at[pl.ds(base, _B_PER_W)], idx_b, sem_ib)
    cp_is.wait()
    cp_ib.wait()

    offs = [sum(_CHUNKS[:k]) for k in range(_NB)]
    cp_s = []
    cp_b = []
    for k, (off, n) in enumerate(zip(offs, _CHUNKS)):
        r = pl.ds(off, n)
        cp_s.append(pltpu.async_copy(
            sess_tab_hbm.at[idx_s.at[r]], rows_s.at[r], sem_s[k]))
        cp_b.append(pltpu.async_copy(
            subj_tab_hbm.at[idx_b.at[r]], rows_b.at[r], sem_b[k]))

    cp_fs.wait()
    cp_fb.wait()
    fs = [flag_s[pl.ds(i * LANES, LANES)] for i in range(DIM // LANES)]
    fb = [flag_b[pl.ds(i * LANES, LANES)] for i in range(DIM // LANES)]

    cp_o = []
    for k, (off, n) in enumerate(zip(offs, _CHUNKS)):
        r = pl.ds(off, n)
        ro = pl.ds(base + off, n)

        cp_s[k].wait()

        @plsc.parallel_loop(off, off + n, unroll=1)
        def _body_s(i):
            for c in range(DIM // LANES):
                d = pl.ds(c * LANES, LANES)
                plsc.addupdate(rows_s.at[i, 0, d], fs[c])

        cp_o.append(pltpu.async_copy(
            rows_s.at[r], out_hbm.at[ro, pl.ds(0, 1)], sem_os[k]))

        cp_b[k].wait()

        @plsc.parallel_loop(off, off + n, unroll=1)
        def _body_b(i):
            for c in range(DIM // LANES):
                d = pl.ds(c * LANES, LANES)
                plsc.addupdate(rows_b.at[i, 0, d], fb[c])

        cp_o.append(pltpu.async_copy(
            rows_b.at[r], out_hbm.at[ro, pl.ds(1, 1)], sem_ob[k]))

    for cp in cp_o:
        cp.wait()


def kernel(session_idx, subject_idx, session_table, subject_table,
           session_flag, subject_flag):
    return _ctx_emb_kernel(
        session_idx, subject_idx,
        session_table.reshape(VOCAB, 1, DIM),
        subject_table.reshape(VOCAB, 1, DIM),
        session_flag, subject_flag)
